# baseline (speedup 1.0000x reference)
import functools

import jax
import jax.numpy as jnp
from jax import lax
from jax.experimental import pallas as pl
from jax.experimental.pallas import tpu as pltpu
from jax.experimental.pallas import tpu_sc as plsc

N = 10000
D = 128
NE = 320000

_info = plsc.get_sparse_core_info()
NC = _info.num_cores
NS = _info.num_subcores
NW = NC * NS
NPT = 320
NPAD = NW * NPT
CHUNK = 6400
HC = CHUNK // 2
NCHUNK = NE // CHUNK
G = 64
ACH = 32
LANES = 16
UNPT = jnp.uint32(NPT)


def _mm_body(x_ref, w_ref, b_ref, a_ref, bm_ref):
    xb = x_ref[...]
    w1 = w_ref[:D, :]
    w2 = w_ref[D:, :]
    bm_ref[...] = jnp.dot(xb, w2, preferred_element_type=jnp.float32)
    a_ref[...] = (
        jnp.dot(xb, w1 - w2, preferred_element_type=jnp.float32) + b_ref[...]
    )


_BLKM = 2048


def _matmuls(xp, W, b2):
    return pl.pallas_call(
        _mm_body,
        grid=(NPAD // _BLKM,),
        in_specs=[
            pl.BlockSpec((_BLKM, D), lambda i: (i, 0)),
            pl.BlockSpec((2 * D, D), lambda i: (0, 0)),
            pl.BlockSpec((1, D), lambda i: (0, 0)),
        ],
        out_specs=[
            pl.BlockSpec((_BLKM, D), lambda i: (i, 0)),
            pl.BlockSpec((_BLKM, D), lambda i: (i, 0)),
        ],
        out_shape=[
            jax.ShapeDtypeStruct((NPAD, D), jnp.float32),
            jax.ShapeDtypeStruct((NPAD, D), jnp.float32),
        ],
    )(xp, W, b2)


def _sc_body(bm_hbm, a_hbm, src_hbm, dst_hbm, out_hbm,
             dstb, srcb, csA, cdA, csB, cdB, accum, rows, abuf, esem, gsem):
    wid = lax.axis_index("s") * NC + lax.axis_index("c")
    lo = wid * NPT

    neg = jnp.full((LANES,), -jnp.inf, jnp.float32)

    def init_body(i, _):
        accum[pl.ds(i * LANES, LANES)] = neg
        return 0

    lax.fori_loop(0, (NPT + 1) * D // LANES, init_body, 0, unroll=8)

    iota = lax.iota(jnp.int32, LANES)
    one = jnp.ones((LANES,), jnp.int32)
    zero16 = jnp.zeros((LANES,), jnp.int32)
    pad16 = jnp.full((LANES,), NPT, jnp.int32)
    nm0 = jnp.zeros((LANES,), jnp.int32)

    def start_chunk(c, buf):
        off = pl.multiple_of(c * CHUNK, CHUNK)
        pltpu.async_copy(dst_hbm.at[pl.ds(off, CHUNK)], dstb.at[buf], esem)
        pltpu.async_copy(src_hbm.at[pl.ds(off, CHUNK)], srcb.at[buf], esem)

    def wait_chunk(buf):
        pltpu.make_async_copy(dst_hbm.at[pl.ds(0, CHUNK)], dstb.at[buf], esem).wait()
        pltpu.make_async_copy(src_hbm.at[pl.ds(0, CHUNK)], srcb.at[buf], esem).wait()

    start_chunk(0, 0)

    def do_stream(cs, cd, nmv):
        for kpad in range(G // LANES):
            posp = nmv + (kpad * LANES) + iota
            plsc.store_scatter(cs, [posp], zero16)
            plsc.store_scatter(cd, [posp], pad16)
        nm = jnp.max(nmv)
        ngrp = (nm + (G - 1)) >> 6

        def fire(g, buf2):
            idx = cs.at[pl.ds(pl.multiple_of(g * G, G), G)]
            pltpu.async_copy(bm_hbm.at[idx], rows.at[buf2], gsem)

        @pl.when(ngrp > 0)
        def _():
            fire(0, 0)

        def grp_body(g, _):
            gbuf = lax.rem(g, 2)
            pltpu.make_async_copy(
                bm_hbm.at[cs.at[pl.ds(0, G)]], rows.at[gbuf], gsem
            ).wait()

            @pl.when(g + 1 < ngrp)
            def _():
                fire(g + 1, 1 - gbuf)

            goff = g * G
            rb0 = cd[pl.ds(goff, LANES)][0] << 7

            def row_body(r, rb):
                rbn = cd[pl.ds(goff + r + 1, LANES)][0] << 7
                avs = [accum[pl.ds(rb + k * LANES, LANES)]
                       for k in range(D // LANES)]
                vvs = [rows[gbuf, r, pl.ds(k * LANES, LANES)]
                       for k in range(D // LANES)]
                for k in range(D // LANES):
                    accum[pl.ds(rb + k * LANES, LANES)] = jnp.maximum(
                        avs[k], vvs[k])
                return rbn

            lax.fori_loop(0, G, row_body, rb0)
            return 0

        lax.fori_loop(0, ngrp, grp_body, 0)

    def chunk_body(c, _):
        buf = lax.rem(c, 2)
        wait_chunk(buf)

        @pl.when(c + 1 < NCHUNK)
        def _():
            start_chunk(c + 1, 1 - buf)

        def scan_body(i, carry):
            nmA, nmB = carry
            base = i * (2 * LANES)
            dA = dstb[buf, pl.ds(base, LANES)]
            dB = dstb[buf, pl.ds(base + LANES, LANES)]
            sA = srcb[buf, pl.ds(base, LANES)]
            sB = srcb[buf, pl.ds(base + LANES, LANES)]
            dlA = dA - lo
            dlB = dB - lo
            mA = dlA.astype(jnp.uint32) < UNPT
            mB = dlB.astype(jnp.uint32) < UNPT
            posA = plsc.cumsum(jnp.where(mA, one, zero16)) + nmA - 1
            posB = plsc.cumsum(jnp.where(mB, one, zero16)) + nmB - 1
            plsc.store_scatter(csA, [posA], sA, mask=mA)
            plsc.store_scatter(cdA, [posA], dlA, mask=mA)
            plsc.store_scatter(csB, [posB], sB, mask=mB)
            plsc.store_scatter(cdB, [posB], dlB, mask=mB)
            return (nmA + plsc.all_reduce_population_count(mA),
                    nmB + plsc.all_reduce_population_count(mB))

        nmA, nmB = lax.fori_loop(0, CHUNK // (2 * LANES), scan_body,
                                 (nm0, nm0), unroll=2)
        do_stream(csA, cdA, nmA)
        do_stream(csB, cdB, nmB)
        return 0

    lax.fori_loop(0, NCHUNK, chunk_body, 0)

    def comb_body(t, _):
        row0 = lo + t * ACH
        pltpu.sync_copy(a_hbm.at[pl.ds(row0, ACH)], abuf)

        def cr(i, _):
            rb = (t * ACH + i) * D
            for k in range(D // LANES):
                av = abuf[i, pl.ds(k * LANES, LANES)]
                mv = accum[pl.ds(rb + k * LANES, LANES)]
                abuf[i, pl.ds(k * LANES, LANES)] = jnp.maximum(av + mv, 0.0)
            return 0

        lax.fori_loop(0, ACH, cr, 0)
        pltpu.sync_copy(abuf, out_hbm.at[pl.ds(row0, ACH)])
        return 0

    lax.fori_loop(0, NPT // ACH, comb_body, 0)


_sc_call = functools.partial(
    pl.kernel,
    out_type=jax.ShapeDtypeStruct((NPAD, D), jnp.float32),
    mesh=plsc.VectorSubcoreMesh(core_axis_name="c", subcore_axis_name="s"),
    scratch_types=[
        pltpu.VMEM((2, CHUNK), jnp.int32),
        pltpu.VMEM((2, CHUNK), jnp.int32),
        pltpu.VMEM((HC + G + LANES,), jnp.int32),
        pltpu.VMEM((HC + G + LANES,), jnp.int32),
        pltpu.VMEM((HC + G + LANES,), jnp.int32),
        pltpu.VMEM((HC + G + LANES,), jnp.int32),
        pltpu.VMEM(((NPT + 1) * D,), jnp.float32),
        pltpu.VMEM((2, G, D), jnp.float32),
        pltpu.VMEM((ACH, D), jnp.float32),
        pltpu.SemaphoreType.DMA,
        pltpu.SemaphoreType.DMA,
    ],
    compiler_params=pltpu.CompilerParams(needs_layout_passes=False),
)(_sc_body)


@jax.jit
def kernel(x, edge_index, W, b):
    ei = edge_index.astype(jnp.int32)
    src = ei[0]
    dst = ei[1]
    xp = jnp.pad(x, ((0, NPAD - N), (0, 0)))
    aprime, bmat = _matmuls(xp, W, b.reshape(1, D))
    out = _sc_call(bmat, aprime, src, dst)
    return out[:N]


# speedup vs baseline: 1.3602x; 1.3602x over previous
import functools

import jax
import jax.numpy as jnp
from jax import lax
from jax.experimental import pallas as pl
from jax.experimental.pallas import tpu as pltpu
from jax.experimental.pallas import tpu_sc as plsc

N = 10000
D = 128
NE = 320000

_info = plsc.get_sparse_core_info()
NC = _info.num_cores
NS = _info.num_subcores
NW = NC * NS
NPT = 320
NPAD = NW * NPT
CHUNK = 6400
NCHUNK = NE // CHUNK
G = 64
ACH = 32
LANES = 16


def _mm_body(x_ref, w_ref, b_ref, a_ref, bm_ref):
    xb = x_ref[...]
    w1 = w_ref[:D, :]
    w2 = w_ref[D:, :]
    bm_ref[...] = jnp.dot(xb, w2, preferred_element_type=jnp.float32)
    a_ref[...] = (
        jnp.dot(xb, w1 - w2, preferred_element_type=jnp.float32) + b_ref[...]
    )


_BLKM = 2048


def _matmuls(xp, W, b2):
    return pl.pallas_call(
        _mm_body,
        grid=(NPAD // _BLKM,),
        in_specs=[
            pl.BlockSpec((_BLKM, D), lambda i: (i, 0)),
            pl.BlockSpec((2 * D, D), lambda i: (0, 0)),
            pl.BlockSpec((1, D), lambda i: (0, 0)),
        ],
        out_specs=[
            pl.BlockSpec((_BLKM, D), lambda i: (i, 0)),
            pl.BlockSpec((_BLKM, D), lambda i: (i, 0)),
        ],
        out_shape=[
            jax.ShapeDtypeStruct((NPAD, D), jnp.float32),
            jax.ShapeDtypeStruct((NPAD, D), jnp.float32),
        ],
    )(xp, W, b2)


def _sc_body(bm_hbm, a_hbm, src_hbm, dst_hbm, out_hbm,
             dstb, srcb, csrc, cdst, accum, rows, abuf, sem):
    wid = lax.axis_index("s") * NC + lax.axis_index("c")
    lo = wid * NPT

    neg = jnp.full((LANES,), -jnp.inf, jnp.float32)

    def init_body(i, _):
        accum[pl.ds(i * LANES, LANES)] = neg
        return 0

    lax.fori_loop(0, (NPT + 1) * D // LANES, init_body, 0, unroll=4)

    iota = lax.iota(jnp.int32, LANES)
    one = jnp.ones((LANES,), jnp.int32)
    zero16 = jnp.zeros((LANES,), jnp.int32)
    pad16 = jnp.full((LANES,), NPT, jnp.int32)

    def chunk_body(c, _):
        off = pl.multiple_of(c * CHUNK, CHUNK)
        pltpu.sync_copy(dst_hbm.at[pl.ds(off, CHUNK)], dstb)
        pltpu.sync_copy(src_hbm.at[pl.ds(off, CHUNK)], srcb)

        def scan_body(i, nm):
            d = dstb[pl.ds(i * LANES, LANES)]
            s = srcb[pl.ds(i * LANES, LANES)]
            dl = d - lo
            m = (dl >= 0) & (dl < NPT)
            pos = plsc.cumsum(jnp.where(m, one, zero16)) + nm - 1
            plsc.store_scatter(csrc, [pos], s, mask=m)
            plsc.store_scatter(cdst, [pos], dl, mask=m)
            return nm + plsc.all_reduce_population_count(m)

        nmv = lax.fori_loop(0, CHUNK // LANES, scan_body,
                            jnp.zeros((LANES,), jnp.int32), unroll=4)
        for kpad in range(G // LANES):
            posp = nmv + (kpad * LANES) + iota
            plsc.store_scatter(csrc, [posp], zero16)
            plsc.store_scatter(cdst, [posp], pad16)
        nm = jnp.max(nmv)
        ngrp = (nm + (G - 1)) >> 6

        def grp_body(g, _):
            goff = g * G
            idx = csrc.at[pl.ds(goff, G)]
            pltpu.async_copy(bm_hbm.at[idx], rows, sem).wait()

            def row_body(r, _):
                ld = cdst[pl.ds(goff + r, LANES)][0]
                rb = ld << 7
                for k in range(D // LANES):
                    a = accum[pl.ds(rb + k * LANES, LANES)]
                    v = rows[r, pl.ds(k * LANES, LANES)]
                    accum[pl.ds(rb + k * LANES, LANES)] = jnp.maximum(a, v)
                return 0

            lax.fori_loop(0, G, row_body, 0)
            return 0

        lax.fori_loop(0, ngrp, grp_body, 0)
        return 0

    lax.fori_loop(0, NCHUNK, chunk_body, 0)

    def comb_body(t, _):
        row0 = lo + t * ACH
        pltpu.sync_copy(a_hbm.at[pl.ds(row0, ACH)], abuf)

        def cr(i, _):
            rb = (t * ACH + i) * D
            for k in range(D // LANES):
                av = abuf[i, pl.ds(k * LANES, LANES)]
                mv = accum[pl.ds(rb + k * LANES, LANES)]
                abuf[i, pl.ds(k * LANES, LANES)] = jnp.maximum(av + mv, 0.0)
            return 0

        lax.fori_loop(0, ACH, cr, 0)
        pltpu.sync_copy(abuf, out_hbm.at[pl.ds(row0, ACH)])
        return 0

    lax.fori_loop(0, NPT // ACH, comb_body, 0)


_sc_call = functools.partial(
    pl.kernel,
    out_type=jax.ShapeDtypeStruct((NPAD, D), jnp.float32),
    mesh=plsc.VectorSubcoreMesh(core_axis_name="c", subcore_axis_name="s"),
    scratch_types=[
        pltpu.VMEM((CHUNK,), jnp.int32),
        pltpu.VMEM((CHUNK,), jnp.int32),
        pltpu.VMEM((CHUNK + G,), jnp.int32),
        pltpu.VMEM((CHUNK + G + LANES,), jnp.int32),
        pltpu.VMEM(((NPT + 1) * D,), jnp.float32),
        pltpu.VMEM((G, D), jnp.float32),
        pltpu.VMEM((ACH, D), jnp.float32),
        pltpu.SemaphoreType.DMA,
    ],
    compiler_params=pltpu.CompilerParams(needs_layout_passes=False),
)(_sc_body)


@jax.jit
def kernel(x, edge_index, W, b):
    ei = edge_index.astype(jnp.int32)
    src = ei[0]
    dst = ei[1]
    xp = jnp.pad(x, ((0, NPAD - N), (0, 0)))
    aprime, bmat = _matmuls(xp, W, b.reshape(1, D))
    out = _sc_call(bmat, aprime, src, dst)
    return out[:N]
